# baseline (speedup 1.0000x reference)
import functools

import jax
import jax.numpy as jnp
from jax import lax
from jax.experimental import pallas as pl
from jax.experimental.pallas import tpu as pltpu
from jax.experimental.pallas import tpu_sc as plsc

N_NODE = 10000
D = 256
NUM_E = 160000
NT = 7
RB = 2000
NE_TOT = 6 * NUM_E


def _transform_body(pro_ref, drug_ref, wppi_ref, wddi_ref, wdpi_ref, cse_ref,
                    out_ref):
    b = pl.program_id(0)
    is_p = jnp.logical_or(b == 0, b == 2)
    x = jnp.where(is_p, pro_ref[...], drug_ref[...])
    eye = (lax.broadcasted_iota(jnp.int32, (D, D), 0)
           == lax.broadcasted_iota(jnp.int32, (D, D), 1)).astype(jnp.float32)
    m = jnp.where(b <= 1, eye,
                  jnp.where(b == 2, wppi_ref[...],
                            jnp.where(b == 3, wdpi_ref[...], wddi_ref[...])))
    c = jnp.where(b == 4, cse_ref[0:1, :],
                  jnp.where(b == 5, cse_ref[1:2, :], cse_ref[2:3, :]))
    cv = jnp.where(b >= 4, c, jnp.ones_like(c))
    out_ref[0] = (jnp.dot(x * cv, m, preferred_element_type=jnp.float32)) * cv


def _build_table(protein_h, drug_h, W_ppi, W_ddi, W_dpi, cse_w):
    grid = (NT, N_NODE // RB)
    out = pl.pallas_call(
        _transform_body,
        grid=grid,
        in_specs=[
            pl.BlockSpec((RB, D), lambda b, r: (r, 0)),
            pl.BlockSpec((RB, D), lambda b, r: (r, 0)),
            pl.BlockSpec((D, D), lambda b, r: (0, 0)),
            pl.BlockSpec((D, D), lambda b, r: (0, 0)),
            pl.BlockSpec((D, D), lambda b, r: (0, 0)),
            pl.BlockSpec((8, D), lambda b, r: (0, 0)),
        ],
        out_specs=pl.BlockSpec((1, RB, D), lambda b, r: (b, r, 0)),
        out_shape=jax.ShapeDtypeStruct((NT, N_NODE, D), jnp.float32),
    )(protein_h, drug_h, W_ppi, W_ddi, W_dpi, cse_w)
    return out.reshape(NT * N_NODE, D)


_INFO = plsc.get_sparse_core_info()
_NC, _NS = _INFO.num_cores, _INFO.num_subcores
_NW = _NC * _NS
_EPW = NE_TOT // _NW
_CB = 240
_NCHUNK = _EPW // _CB


def _score_kernel(table_hbm, src_hbm, dst_hbm, out_hbm,
                  idx_s_v, idx_d_v, rows_s, rows_d, scores_v, sem_s, sem_d):
    wid = lax.axis_index("s") * _NC + lax.axis_index("c")
    base = wid * _EPW

    def chunk_body(c, _):
        off = base + c * _CB
        pltpu.sync_copy(src_hbm.at[pl.ds(off, _CB)], idx_s_v)
        pltpu.sync_copy(dst_hbm.at[pl.ds(off, _CB)], idx_d_v)
        cp_s = pltpu.async_copy(table_hbm.at[idx_s_v], rows_s, sem_s)
        cp_d = pltpu.async_copy(table_hbm.at[idx_d_v], rows_d, sem_d)
        cp_s.wait()
        cp_d.wait()

        def group_body(g, _):
            row_ids = g * 16 + lax.iota(jnp.int32, 16)
            acc = jnp.zeros((16,), jnp.float32)
            for j in range(D):
                col = jnp.full((16,), j, jnp.int32)
                a = plsc.load_gather(rows_s, [row_ids, col])
                bb = plsc.load_gather(rows_d, [row_ids, col])
                acc = acc + a * bb
            scores_v[pl.ds(g * 16, 16)] = acc
            return 0

        lax.fori_loop(0, _CB // 16, group_body, 0)
        pltpu.sync_copy(scores_v, out_hbm.at[pl.ds(off, _CB)])
        return 0

    lax.fori_loop(0, _NCHUNK, chunk_body, 0)


def _score(table, src_idx, dst_idx):
    mesh = plsc.VectorSubcoreMesh(core_axis_name="c", subcore_axis_name="s")
    run = pl.kernel(
        _score_kernel,
        out_type=jax.ShapeDtypeStruct((NE_TOT,), jnp.float32),
        mesh=mesh,
        scratch_types=[
            pltpu.VMEM((_CB,), jnp.int32),
            pltpu.VMEM((_CB,), jnp.int32),
            pltpu.VMEM((_CB, D), jnp.float32),
            pltpu.VMEM((_CB, D), jnp.float32),
            pltpu.VMEM((_CB,), jnp.float32),
            pltpu.SemaphoreType.DMA,
            pltpu.SemaphoreType.DMA,
        ],
        compiler_params=pltpu.CompilerParams(use_tc_tiling_on_sc=False,
                                             needs_layout_passes=False),
    )
    return run(table, src_idx, dst_idx)


def kernel(drug_h, protein_h, ppi_edge_index, dpi_edge_index, pdi_edge_index,
           ddi_edge_index, W_ppi, W_ddi, W_dpi, cse_w):
    cse_pad = jnp.zeros((8, D), jnp.float32).at[:3].set(cse_w)
    table = _build_table(protein_h, drug_h, W_ppi, W_ddi, W_dpi, cse_pad)

    i32 = jnp.int32
    src_idx = jnp.concatenate([
        ppi_edge_index[0].astype(i32) + 2 * N_NODE,
        dpi_edge_index[0].astype(i32) + 3 * N_NODE,
        pdi_edge_index[0].astype(i32),
        ddi_edge_index[0, 0].astype(i32) + 4 * N_NODE,
        ddi_edge_index[1, 0].astype(i32) + 5 * N_NODE,
        ddi_edge_index[2, 0].astype(i32) + 6 * N_NODE,
    ])
    dst_idx = jnp.concatenate([
        ppi_edge_index[1].astype(i32),
        dpi_edge_index[1].astype(i32),
        pdi_edge_index[1].astype(i32) + 3 * N_NODE,
        ddi_edge_index[0, 1].astype(i32) + N_NODE,
        ddi_edge_index[1, 1].astype(i32) + N_NODE,
        ddi_edge_index[2, 1].astype(i32) + N_NODE,
    ])

    scores = _score(table, src_idx, dst_idx)
    return scores.reshape(6, NUM_E)


# speedup vs baseline: 4.7533x; 4.7533x over previous
import functools

import jax
import jax.numpy as jnp
from jax import lax
from jax.experimental import pallas as pl
from jax.experimental.pallas import tpu as pltpu
from jax.experimental.pallas import tpu_sc as plsc

N_NODE = 10000
D = 256
NUM_E = 160000
NT = 7
RB = 2000
NE_TOT = 6 * NUM_E


def _transform_body(pro_ref, drug_ref, wppi_ref, wddi_ref, wdpi_ref, cse_ref,
                    out_ref):
    b = pl.program_id(0)
    is_p = jnp.logical_or(b == 0, b == 2)
    x = jnp.where(is_p, pro_ref[...], drug_ref[...])
    eye = (lax.broadcasted_iota(jnp.int32, (D, D), 0)
           == lax.broadcasted_iota(jnp.int32, (D, D), 1)).astype(jnp.float32)
    m = jnp.where(b <= 1, eye,
                  jnp.where(b == 2, wppi_ref[...],
                            jnp.where(b == 3, wdpi_ref[...], wddi_ref[...])))
    c = jnp.where(b == 4, cse_ref[0:1, :],
                  jnp.where(b == 5, cse_ref[1:2, :], cse_ref[2:3, :]))
    cv = jnp.where(b >= 4, c, jnp.ones_like(c))
    out_ref[0] = (jnp.dot(x * cv, m, preferred_element_type=jnp.float32)) * cv


def _build_table(protein_h, drug_h, W_ppi, W_ddi, W_dpi, cse_w):
    grid = (NT, N_NODE // RB)
    out = pl.pallas_call(
        _transform_body,
        grid=grid,
        in_specs=[
            pl.BlockSpec((RB, D), lambda b, r: (r, 0)),
            pl.BlockSpec((RB, D), lambda b, r: (r, 0)),
            pl.BlockSpec((D, D), lambda b, r: (0, 0)),
            pl.BlockSpec((D, D), lambda b, r: (0, 0)),
            pl.BlockSpec((D, D), lambda b, r: (0, 0)),
            pl.BlockSpec((8, D), lambda b, r: (0, 0)),
        ],
        out_specs=pl.BlockSpec((1, RB, D), lambda b, r: (b, r, 0)),
        out_shape=jax.ShapeDtypeStruct((NT, N_NODE, D), jnp.float32),
    )(protein_h, drug_h, W_ppi, W_ddi, W_dpi, cse_w)
    return out.reshape(NT * N_NODE, D)


_INFO = plsc.get_sparse_core_info()
_NC, _NS = _INFO.num_cores, _INFO.num_subcores
_NW = _NC * _NS
_EPW = NE_TOT // _NW
_CB = 240
_NCHUNK = _EPW // _CB


def _score_kernel(table_hbm, src_hbm, dst_hbm, out_hbm,
                  idx_s_v, idx_d_v, rows_s, rows_d, scores_v, tmp_v,
                  sem_s, sem_d):
    wid = lax.axis_index("s") * _NC + lax.axis_index("c")
    base = wid * _EPW

    def chunk_body(c, _):
        off = base + c * _CB
        pltpu.sync_copy(src_hbm.at[pl.ds(off, _CB)], idx_s_v)
        pltpu.sync_copy(dst_hbm.at[pl.ds(off, _CB)], idx_d_v)
        cp_s = pltpu.async_copy(table_hbm.at[idx_s_v], rows_s, sem_s)
        cp_d = pltpu.async_copy(table_hbm.at[idx_d_v], rows_d, sem_d)
        cp_s.wait()
        cp_d.wait()

        def group_body(g, _):
            e0 = g * 16
            for l in range(16):
                e = e0 + l
                acc = rows_s[e, pl.ds(0, 16)] * rows_d[e, pl.ds(0, 16)]
                for cc in range(1, D // 16):
                    acc = acc + (rows_s[e, pl.ds(cc * 16, 16)]
                                 * rows_d[e, pl.ds(cc * 16, 16)])
                tmp_v[l, pl.ds(0, 16)] = acc
            lane = lax.iota(jnp.int32, 16)
            score = plsc.load_gather(tmp_v, [lane, jnp.zeros((16,), jnp.int32)])
            for cc in range(1, 16):
                score = score + plsc.load_gather(
                    tmp_v, [lane, jnp.full((16,), cc, jnp.int32)])
            scores_v[pl.ds(e0, 16)] = score
            return 0

        lax.fori_loop(0, _CB // 16, group_body, 0)
        pltpu.sync_copy(scores_v, out_hbm.at[pl.ds(off, _CB)])
        return 0

    lax.fori_loop(0, _NCHUNK, chunk_body, 0)


def _score(table, src_idx, dst_idx):
    mesh = plsc.VectorSubcoreMesh(core_axis_name="c", subcore_axis_name="s")
    run = pl.kernel(
        _score_kernel,
        out_type=jax.ShapeDtypeStruct((NE_TOT,), jnp.float32),
        mesh=mesh,
        scratch_types=[
            pltpu.VMEM((_CB,), jnp.int32),
            pltpu.VMEM((_CB,), jnp.int32),
            pltpu.VMEM((_CB, D), jnp.float32),
            pltpu.VMEM((_CB, D), jnp.float32),
            pltpu.VMEM((_CB,), jnp.float32),
            pltpu.VMEM((16, 17), jnp.float32),
            pltpu.SemaphoreType.DMA,
            pltpu.SemaphoreType.DMA,
        ],
        compiler_params=pltpu.CompilerParams(use_tc_tiling_on_sc=False,
                                             needs_layout_passes=False),
    )
    return run(table, src_idx, dst_idx)


def kernel(drug_h, protein_h, ppi_edge_index, dpi_edge_index, pdi_edge_index,
           ddi_edge_index, W_ppi, W_ddi, W_dpi, cse_w):
    cse_pad = jnp.zeros((8, D), jnp.float32).at[:3].set(cse_w)
    table = _build_table(protein_h, drug_h, W_ppi, W_ddi, W_dpi, cse_pad)

    i32 = jnp.int32
    src_idx = jnp.concatenate([
        ppi_edge_index[0].astype(i32) + 2 * N_NODE,
        dpi_edge_index[0].astype(i32) + 3 * N_NODE,
        pdi_edge_index[0].astype(i32),
        ddi_edge_index[0, 0].astype(i32) + 4 * N_NODE,
        ddi_edge_index[1, 0].astype(i32) + 5 * N_NODE,
        ddi_edge_index[2, 0].astype(i32) + 6 * N_NODE,
    ])
    dst_idx = jnp.concatenate([
        ppi_edge_index[1].astype(i32),
        dpi_edge_index[1].astype(i32),
        pdi_edge_index[1].astype(i32) + 3 * N_NODE,
        ddi_edge_index[0, 1].astype(i32) + N_NODE,
        ddi_edge_index[1, 1].astype(i32) + N_NODE,
        ddi_edge_index[2, 1].astype(i32) + N_NODE,
    ])

    scores = _score(table, src_idx, dst_idx)
    return scores.reshape(6, NUM_E)


# speedup vs baseline: 6.1175x; 1.2870x over previous
import functools

import jax
import jax.numpy as jnp
from jax import lax
from jax.experimental import pallas as pl
from jax.experimental.pallas import tpu as pltpu
from jax.experimental.pallas import tpu_sc as plsc

N_NODE = 10000
D = 256
NUM_E = 160000
NT = 7
RB = 2000
NE_TOT = 6 * NUM_E


def _transform_body(pro_ref, drug_ref, wppi_ref, wddi_ref, wdpi_ref, cse_ref,
                    out_ref):
    b = pl.program_id(0)
    is_p = jnp.logical_or(b == 0, b == 2)
    x = jnp.where(is_p, pro_ref[...], drug_ref[...])
    eye = (lax.broadcasted_iota(jnp.int32, (D, D), 0)
           == lax.broadcasted_iota(jnp.int32, (D, D), 1)).astype(jnp.float32)
    m = jnp.where(b <= 1, eye,
                  jnp.where(b == 2, wppi_ref[...],
                            jnp.where(b == 3, wdpi_ref[...], wddi_ref[...])))
    c = jnp.where(b == 4, cse_ref[0:1, :],
                  jnp.where(b == 5, cse_ref[1:2, :], cse_ref[2:3, :]))
    cv = jnp.where(b >= 4, c, jnp.ones_like(c))
    y = (jnp.dot(x * cv, m, preferred_element_type=jnp.float32)) * cv
    out_ref[0] = y.astype(jnp.bfloat16)


def _build_table(protein_h, drug_h, W_ppi, W_ddi, W_dpi, cse_w):
    grid = (NT, N_NODE // RB)
    out = pl.pallas_call(
        _transform_body,
        grid=grid,
        in_specs=[
            pl.BlockSpec((RB, D), lambda b, r: (r, 0)),
            pl.BlockSpec((RB, D), lambda b, r: (r, 0)),
            pl.BlockSpec((D, D), lambda b, r: (0, 0)),
            pl.BlockSpec((D, D), lambda b, r: (0, 0)),
            pl.BlockSpec((D, D), lambda b, r: (0, 0)),
            pl.BlockSpec((8, D), lambda b, r: (0, 0)),
        ],
        out_specs=pl.BlockSpec((1, RB, D), lambda b, r: (b, r, 0)),
        out_shape=jax.ShapeDtypeStruct((NT, N_NODE, D), jnp.bfloat16),
    )(protein_h, drug_h, W_ppi, W_ddi, W_dpi, cse_w)
    return out.reshape(NT * N_NODE, D)


_INFO = plsc.get_sparse_core_info()
_NC, _NS = _INFO.num_cores, _INFO.num_subcores
_NW = _NC * _NS
_EPW = NE_TOT // _NW
_CB = 240
_NCHUNK = _EPW // _CB


def _score_kernel(table_hbm, src_hbm, dst_hbm, out_hbm,
                  idx_s_v, idx_d_v, rows_s, rows_d, scores_v, tmp_v,
                  sem_s, sem_d):
    wid = lax.axis_index("s") * _NC + lax.axis_index("c")
    base = wid * _EPW

    def chunk_body(c, _):
        off = base + c * _CB
        pltpu.sync_copy(src_hbm.at[pl.ds(off, _CB)], idx_s_v)
        pltpu.sync_copy(dst_hbm.at[pl.ds(off, _CB)], idx_d_v)
        cp_s = pltpu.async_copy(table_hbm.at[idx_s_v], rows_s, sem_s)
        cp_d = pltpu.async_copy(table_hbm.at[idx_d_v], rows_d, sem_d)
        cp_s.wait()
        cp_d.wait()

        def group_body(g, _):
            e0 = g * 16
            for l in range(16):
                e = e0 + l
                acc_a = jnp.zeros((16,), jnp.float32)
                acc_b = jnp.zeros((16,), jnp.float32)
                for cc in range(D // 32):
                    ws = rows_s[e, pl.ds(cc * 32, 32)]
                    wd = rows_d[e, pl.ds(cc * 32, 32)]
                    sa, sb = plsc.unpack(ws, format=plsc.PackFormat.INTERLEAVED)
                    da, db = plsc.unpack(wd, format=plsc.PackFormat.INTERLEAVED)
                    acc_a = acc_a + sa * da
                    acc_b = acc_b + sb * db
                tmp_v[l, pl.ds(0, 16)] = acc_a + acc_b
            lane = lax.iota(jnp.int32, 16)
            score = plsc.load_gather(tmp_v, [lane, jnp.zeros((16,), jnp.int32)])
            for cc in range(1, 16):
                score = score + plsc.load_gather(
                    tmp_v, [lane, jnp.full((16,), cc, jnp.int32)])
            scores_v[pl.ds(e0, 16)] = score
            return 0

        lax.fori_loop(0, _CB // 16, group_body, 0)
        pltpu.sync_copy(scores_v, out_hbm.at[pl.ds(off, _CB)])
        return 0

    lax.fori_loop(0, _NCHUNK, chunk_body, 0)


def _score(table, src_idx, dst_idx):
    mesh = plsc.VectorSubcoreMesh(core_axis_name="c", subcore_axis_name="s")
    run = pl.kernel(
        _score_kernel,
        out_type=jax.ShapeDtypeStruct((NE_TOT,), jnp.float32),
        mesh=mesh,
        scratch_types=[
            pltpu.VMEM((_CB,), jnp.int32),
            pltpu.VMEM((_CB,), jnp.int32),
            pltpu.VMEM((_CB, D), jnp.bfloat16),
            pltpu.VMEM((_CB, D), jnp.bfloat16),
            pltpu.VMEM((_CB,), jnp.float32),
            pltpu.VMEM((16, 17), jnp.float32),
            pltpu.SemaphoreType.DMA,
            pltpu.SemaphoreType.DMA,
        ],
        compiler_params=pltpu.CompilerParams(use_tc_tiling_on_sc=False,
                                             needs_layout_passes=False),
    )
    return run(table, src_idx, dst_idx)


def kernel(drug_h, protein_h, ppi_edge_index, dpi_edge_index, pdi_edge_index,
           ddi_edge_index, W_ppi, W_ddi, W_dpi, cse_w):
    cse_pad = jnp.zeros((8, D), jnp.float32).at[:3].set(cse_w)
    table = _build_table(protein_h, drug_h, W_ppi, W_ddi, W_dpi, cse_pad)

    i32 = jnp.int32
    src_idx = jnp.concatenate([
        ppi_edge_index[0].astype(i32) + 2 * N_NODE,
        dpi_edge_index[0].astype(i32) + 3 * N_NODE,
        pdi_edge_index[0].astype(i32),
        ddi_edge_index[0, 0].astype(i32) + 4 * N_NODE,
        ddi_edge_index[1, 0].astype(i32) + 5 * N_NODE,
        ddi_edge_index[2, 0].astype(i32) + 6 * N_NODE,
    ])
    dst_idx = jnp.concatenate([
        ppi_edge_index[1].astype(i32),
        dpi_edge_index[1].astype(i32),
        pdi_edge_index[1].astype(i32) + 3 * N_NODE,
        ddi_edge_index[0, 1].astype(i32) + N_NODE,
        ddi_edge_index[1, 1].astype(i32) + N_NODE,
        ddi_edge_index[2, 1].astype(i32) + N_NODE,
    ])

    scores = _score(table, src_idx, dst_idx)
    return scores.reshape(6, NUM_E)
